# baseline (speedup 1.0000x reference)
import functools

import jax
import jax.numpy as jnp
from jax import lax
from jax.experimental import pallas as pl
from jax.experimental.pallas import tpu as pltpu
from jax.experimental.pallas import tpu_sc as plsc

N_ROWS = 320000
D = 128
NCLS = 3
NC = 2
NS = 16
NW = NC * NS
CH = 400
NCHUNK = 10
RPW = CH * NCHUNK
N_SC = NW * RPW
GRP = CH // 16

TB = 3200
N_TC = N_ROWS - N_SC
TC_OFF = N_SC // TB
TC_BLOCKS = N_TC // TB


def _sc_body(feat_hbm, lab_hbm, psum_hbm, buf_v, labs_v, accv, sem_f, sem_l):
  c = lax.axis_index("c")
  s = lax.axis_index("s")
  wid = s * NC + c
  base = wid * RPW

  pltpu.async_copy(lab_hbm.at[pl.ds(base, RPW)], labs_v, sem_l).wait()
  pltpu.async_copy(feat_hbm.at[pl.ds(base, CH)],
                   buf_v.at[pl.ds(0, CH)], sem_f)

  zf = jnp.zeros((16,), jnp.float32)
  for i in range(NCLS * D // 16):
    accv[pl.ds(i * 16, 16)] = zf

  def chunk_body(g, carry):
    @pl.when(g + 1 < NCHUNK)
    def _():
      nslot = lax.rem(g + 1, 2)
      pltpu.async_copy(
          feat_hbm.at[pl.ds(base + (g + 1) * CH, CH)],
          buf_v.at[pl.ds(nslot * CH, CH)], sem_f)

    pltpu.make_async_copy(feat_hbm.at[pl.ds(0, CH)],
                          buf_v.at[pl.ds(0, CH)], sem_f).wait()
    rowoff = lax.rem(g, 2) * CH
    labbase = g * CH

    def grp_body(t, cr):
      lab_vec = labs_v[pl.ds(labbase + t * 16, 16)]
      m0v = (1 - jnp.minimum(lab_vec, 1)).astype(jnp.float32)
      m1v = (lab_vec & 1).astype(jnp.float32)
      tot = [None] * 8
      a0 = [None] * 8
      a1 = [None] * 8
      for j in range(16):
        idxj = jnp.full((16,), j, jnp.int32)
        m0 = jnp.take_along_axis(m0v, idxj, axis=0, mode="promise_in_bounds")
        m1 = jnp.take_along_axis(m1v, idxj, axis=0, mode="promise_in_bounds")
        r = rowoff + t * 16 + j
        for cc in range(8):
          x = buf_v[r, pl.ds(cc * 16, 16)]
          if j == 0:
            tot[cc] = x
            a0[cc] = x * m0
            a1[cc] = x * m1
          else:
            tot[cc] = tot[cc] + x
            a0[cc] = a0[cc] + x * m0
            a1[cc] = a1[cc] + x * m1
      for cc in range(8):
        plsc.addupdate(accv.at[pl.ds(cc * 16, 16)], a0[cc])
        plsc.addupdate(accv.at[pl.ds(D + cc * 16, 16)], a1[cc])
        plsc.addupdate(accv.at[pl.ds(2 * D + cc * 16, 16)],
                       tot[cc] - a0[cc] - a1[cc])
      return cr

    return plsc.parallel_loop(0, GRP, 1, unroll=2, carry=carry)(grp_body)

  lax.fori_loop(0, NCHUNK, chunk_body, 0)

  pltpu.sync_copy(accv, psum_hbm.at[wid])


@functools.partial(
    pl.kernel,
    out_type=(jax.ShapeDtypeStruct((NW, NCLS * D), jnp.float32),),
    mesh=plsc.VectorSubcoreMesh(core_axis_name="c", subcore_axis_name="s"),
    scratch_types=[
        pltpu.VMEM((2 * CH, D), jnp.float32),
        pltpu.VMEM((RPW,), jnp.int32),
        pltpu.VMEM((NCLS * D,), jnp.float32),
        pltpu.SemaphoreType.DMA,
        pltpu.SemaphoreType.DMA,
    ],
)
def _sc_partials(*args):
  _sc_body(*args)


def _tc_body(lab_ref, feat_ref, out_ref):
  i = pl.program_id(0)
  labs = lab_ref[0, 0, :]
  oh = (labs[None, :] == lax.broadcasted_iota(jnp.int32, (NCLS, TB), 0))
  oh = oh.astype(jnp.float32)
  part = lax.dot_general(oh, feat_ref[...], (((1,), (0,)), ((), ())),
                         preferred_element_type=jnp.float32)

  @pl.when(i == 0)
  def _():
    out_ref[...] = jnp.zeros_like(out_ref)

  out_ref[...] += part


def _finish_body(ps_ref, tc_ref, out_ref):
  rows = [tc_ref[k, :] + jnp.sum(ps_ref[:, k * D:(k + 1) * D], axis=0)
          for k in range(NCLS)]
  sums = jnp.stack(rows, axis=0)
  nrm = jnp.sqrt(jnp.sum(sums * sums, axis=1, keepdims=True))
  out_ref[...] = sums / jnp.maximum(nrm, 1e-12)


def kernel(features, labels):
  (psums,) = _sc_partials(features, labels)
  labs3 = labels.reshape(N_ROWS // TB, 1, TB)
  tc_sums = pl.pallas_call(
      _tc_body,
      grid=(TC_BLOCKS,),
      in_specs=[
          pl.BlockSpec((1, 1, TB), lambda i: (TC_OFF + i, 0, 0)),
          pl.BlockSpec((TB, D), lambda i: (TC_OFF + i, 0)),
      ],
      out_specs=pl.BlockSpec((NCLS, D), lambda i: (0, 0)),
      out_shape=jax.ShapeDtypeStruct((NCLS, D), jnp.float32),
  )(labs3, features)
  fea_center = pl.pallas_call(
      _finish_body,
      out_shape=jax.ShapeDtypeStruct((NCLS, D), jnp.float32),
  )(psums, tc_sums)
  target = jnp.arange(NCLS, dtype=jnp.int32)
  return (fea_center, target)


# speedup vs baseline: 1.0962x; 1.0962x over previous
import functools

import jax
import jax.numpy as jnp
from jax import lax
from jax.experimental import pallas as pl
from jax.experimental.pallas import tpu as pltpu
from jax.experimental.pallas import tpu_sc as plsc

N_ROWS = 320000
D = 128
NCLS = 3
NC = 2
NS = 16
NW = NC * NS
CH = 400
NCHUNK = 13
RPW = CH * NCHUNK
N_SC = NW * RPW
GRP = CH // 16

TB = 3200
N_TC = N_ROWS - N_SC
TC_OFF = N_SC // TB
TC_BLOCKS = N_TC // TB


def _sc_body(feat_hbm, lab_hbm, psum_hbm, buf_v, labs_v, accv, sem_f, sem_l):
  c = lax.axis_index("c")
  s = lax.axis_index("s")
  wid = s * NC + c
  base = wid * RPW

  pltpu.async_copy(lab_hbm.at[pl.ds(base, RPW)], labs_v, sem_l).wait()
  pltpu.async_copy(feat_hbm.at[pl.ds(base, CH)],
                   buf_v.at[pl.ds(0, CH)], sem_f)

  zf = jnp.zeros((16,), jnp.float32)
  for i in range(NCLS * D // 16):
    accv[pl.ds(i * 16, 16)] = zf

  def chunk_body(g, carry):
    @pl.when(g + 1 < NCHUNK)
    def _():
      nslot = lax.rem(g + 1, 2)
      pltpu.async_copy(
          feat_hbm.at[pl.ds(base + (g + 1) * CH, CH)],
          buf_v.at[pl.ds(nslot * CH, CH)], sem_f)

    pltpu.make_async_copy(feat_hbm.at[pl.ds(0, CH)],
                          buf_v.at[pl.ds(0, CH)], sem_f).wait()
    rowoff = lax.rem(g, 2) * CH
    labbase = g * CH

    def grp_body(t, cr):
      lab_vec = labs_v[pl.ds(labbase + t * 16, 16)]
      m0v = (1 - jnp.minimum(lab_vec, 1)).astype(jnp.float32)
      m1v = (lab_vec & 1).astype(jnp.float32)
      tot = [None] * 8
      a0 = [None] * 8
      a1 = [None] * 8
      for j in range(16):
        idxj = jnp.full((16,), j, jnp.int32)
        m0 = jnp.take_along_axis(m0v, idxj, axis=0, mode="promise_in_bounds")
        m1 = jnp.take_along_axis(m1v, idxj, axis=0, mode="promise_in_bounds")
        r = rowoff + t * 16 + j
        for cc in range(8):
          x = buf_v[r, pl.ds(cc * 16, 16)]
          if j == 0:
            tot[cc] = x
            a0[cc] = x * m0
            a1[cc] = x * m1
          else:
            tot[cc] = tot[cc] + x
            a0[cc] = a0[cc] + x * m0
            a1[cc] = a1[cc] + x * m1
      for cc in range(8):
        plsc.addupdate(accv.at[pl.ds(cc * 16, 16)], a0[cc])
        plsc.addupdate(accv.at[pl.ds(D + cc * 16, 16)], a1[cc])
        plsc.addupdate(accv.at[pl.ds(2 * D + cc * 16, 16)],
                       tot[cc] - a0[cc] - a1[cc])
      return cr

    return plsc.parallel_loop(0, GRP, 1, unroll=2, carry=carry)(grp_body)

  lax.fori_loop(0, NCHUNK, chunk_body, 0)

  pltpu.sync_copy(accv, psum_hbm.at[wid])


@functools.partial(
    pl.kernel,
    out_type=(jax.ShapeDtypeStruct((NW, NCLS * D), jnp.float32),),
    mesh=plsc.VectorSubcoreMesh(core_axis_name="c", subcore_axis_name="s"),
    scratch_types=[
        pltpu.VMEM((2 * CH, D), jnp.float32),
        pltpu.VMEM((RPW,), jnp.int32),
        pltpu.VMEM((NCLS * D,), jnp.float32),
        pltpu.SemaphoreType.DMA,
        pltpu.SemaphoreType.DMA,
    ],
)
def _sc_partials(*args):
  _sc_body(*args)


def _tc_body(lab_ref, feat_ref, out_ref):
  i = pl.program_id(0)
  labs = lab_ref[0, 0, :]
  oh = (labs[None, :] == lax.broadcasted_iota(jnp.int32, (NCLS, TB), 0))
  oh = oh.astype(jnp.float32)
  part = lax.dot_general(oh, feat_ref[...], (((1,), (0,)), ((), ())),
                         preferred_element_type=jnp.float32)

  @pl.when(i == 0)
  def _():
    out_ref[...] = jnp.zeros_like(out_ref)

  out_ref[...] += part


def _finish_body(ps_ref, tc_ref, out_ref):
  rows = [tc_ref[k, :] + jnp.sum(ps_ref[:, k * D:(k + 1) * D], axis=0)
          for k in range(NCLS)]
  sums = jnp.stack(rows, axis=0)
  nrm = jnp.sqrt(jnp.sum(sums * sums, axis=1, keepdims=True))
  out_ref[...] = sums / jnp.maximum(nrm, 1e-12)


def kernel(features, labels):
  (psums,) = _sc_partials(features, labels)
  labs3 = labels.reshape(N_ROWS // TB, 1, TB)
  tc_sums = pl.pallas_call(
      _tc_body,
      grid=(TC_BLOCKS,),
      in_specs=[
          pl.BlockSpec((1, 1, TB), lambda i: (TC_OFF + i, 0, 0)),
          pl.BlockSpec((TB, D), lambda i: (TC_OFF + i, 0)),
      ],
      out_specs=pl.BlockSpec((NCLS, D), lambda i: (0, 0)),
      out_shape=jax.ShapeDtypeStruct((NCLS, D), jnp.float32),
  )(labs3, features)
  fea_center = pl.pallas_call(
      _finish_body,
      out_shape=jax.ShapeDtypeStruct((NCLS, D), jnp.float32),
  )(psums, tc_sums)
  target = jnp.arange(NCLS, dtype=jnp.int32)
  return (fea_center, target)
